# baseline (speedup 1.0000x reference)
import jax
import jax.numpy as jnp
from jax import lax
from jax.experimental import pallas as pl
from jax.experimental.pallas import tpu as pltpu
from jax.experimental.pallas import tpu_sc as plsc

_B = 16384
_D = 32
_DW = _D + 1
_NF = 26
_NW = 32
_BPW = _B // _NW
_R = 2048


def _sc_gather_body(ul, il, uidx, iidx, u_out, i_out,
                    uidx_v, iidx_v, urows_v, irows_v, sem):
    wid = lax.axis_index("s") * 2 + lax.axis_index("c")
    base = wid * _BPW
    pltpu.sync_copy(uidx.at[pl.ds(base, _BPW)], uidx_v)
    pltpu.sync_copy(iidx.at[pl.ds(base, _BPW)], iidx_v)
    cu = pltpu.async_copy(ul.at[uidx_v], urows_v, sem)
    ci = pltpu.async_copy(il.at[iidx_v], irows_v, sem)
    cu.wait()
    ci.wait()
    pltpu.sync_copy(urows_v, u_out.at[pl.ds(base, _BPW)])
    pltpu.sync_copy(irows_v, i_out.at[pl.ds(base, _BPW)])


_sc_gather = pl.kernel(
    _sc_gather_body,
    mesh=plsc.VectorSubcoreMesh(core_axis_name="c", subcore_axis_name="s"),
    out_type=[
        jax.ShapeDtypeStruct((_B, _DW), jnp.float32),
        jax.ShapeDtypeStruct((_B, _DW), jnp.float32),
    ],
    scratch_types=[
        pltpu.VMEM((_BPW,), jnp.int32),
        pltpu.VMEM((_BPW,), jnp.int32),
        pltpu.VMEM((_BPW, _DW), jnp.float32),
        pltpu.VMEM((_BPW, _DW), jnp.float32),
        pltpu.SemaphoreType.DMA,
    ],
    compiler_params=pltpu.CompilerParams(use_tc_tiling_on_sc=False),
)


def _tc_combine_body(feats_ref, u_ref, i_ref, fl_ref, fw_ref, fb_ref,
                     out_ref):
    f = feats_ref[...]
    uall = u_ref[...]
    iall = i_ref[...]
    u = uall[:, :_D]
    iv = iall[:, :_D]
    uw = uall[:, _D:]
    iw = iall[:, _D:]
    w = fw_ref[...]
    p = lax.dot_general(u, fl_ref[...], (((1,), (1,)), ((), ())),
                        preferred_element_type=jnp.float32)
    lin = jnp.sum(f * w, axis=1, keepdims=True)
    inter1 = jnp.sum(u * iv, axis=1, keepdims=True)
    inter2 = jnp.sum(p * f, axis=1, keepdims=True)
    out_ref[...] = lin + fb_ref[0, 0] + uw + iw + inter1 + inter2


def _tc_combine(feats, u_e, i_e, fl, fw, fb):
    nblk = _B // _R
    return pl.pallas_call(
        _tc_combine_body,
        grid=(nblk,),
        in_specs=[
            pl.BlockSpec((_R, _NF), lambda i: (i, 0)),
            pl.BlockSpec((_R, _DW), lambda i: (i, 0)),
            pl.BlockSpec((_R, _DW), lambda i: (i, 0)),
            pl.BlockSpec((_NF, _D), lambda i: (0, 0)),
            pl.BlockSpec((1, _NF), lambda i: (0, 0)),
            pl.BlockSpec((1, 1), lambda i: (0, 0)),
        ],
        out_specs=pl.BlockSpec((_R, 1), lambda i: (i, 0)),
        out_shape=jax.ShapeDtypeStruct((_B, 1), jnp.float32),
    )(feats, u_e, i_e, fl, fw, fb)


def kernel(x, user_latent, item_latent, feat_latent, fw_W, fw_b,
           user_weight, item_weight):
    users = x[:, 0].astype(jnp.int32)
    items = x[:, 1].astype(jnp.int32)
    feats = x[:, 2:]
    ucat = jnp.concatenate([user_latent, user_weight], axis=1)
    icat = jnp.concatenate([item_latent, item_weight], axis=1)
    u_e, i_e = _sc_gather(ucat, icat, users, items)
    return _tc_combine(feats, u_e, i_e,
                       feat_latent, fw_W, jnp.reshape(fw_b, (1, 1)))


# speedup vs baseline: 1.9690x; 1.9690x over previous
import jax
import jax.numpy as jnp
from jax import lax
from jax.experimental import pallas as pl
from jax.experimental.pallas import tpu as pltpu
from jax.experimental.pallas import tpu_sc as plsc

_B = 16384
_D = 32
_NF = 26
_NW = 32
_BPW = _B // _NW
_R = 2048


def _sc_gather_body(lat, wt, uidx, iidx, u_out, i_out, uw_out, iw_out,
                    uidx_v, iidx_v, urows_v, irows_v, uw_v, iw_v, sem):
    wid = lax.axis_index("s") * 2 + lax.axis_index("c")
    base = wid * _BPW
    pltpu.sync_copy(uidx.at[pl.ds(base, _BPW)], uidx_v)
    pltpu.sync_copy(iidx.at[pl.ds(base, _BPW)], iidx_v)
    c0 = pltpu.async_copy(lat.at[uidx_v], urows_v, sem)
    c1 = pltpu.async_copy(lat.at[iidx_v], irows_v, sem)
    c2 = pltpu.async_copy(wt.at[uidx_v], uw_v, sem)
    c3 = pltpu.async_copy(wt.at[iidx_v], iw_v, sem)
    c0.wait()
    c1.wait()
    c2.wait()
    c3.wait()
    pltpu.sync_copy(urows_v, u_out.at[pl.ds(base, _BPW)])
    pltpu.sync_copy(irows_v, i_out.at[pl.ds(base, _BPW)])
    pltpu.sync_copy(uw_v, uw_out.at[pl.ds(base, _BPW)])
    pltpu.sync_copy(iw_v, iw_out.at[pl.ds(base, _BPW)])


_sc_gather = pl.kernel(
    _sc_gather_body,
    mesh=plsc.VectorSubcoreMesh(core_axis_name="c", subcore_axis_name="s"),
    out_type=[
        jax.ShapeDtypeStruct((_B, _D), jnp.float32),
        jax.ShapeDtypeStruct((_B, _D), jnp.float32),
        jax.ShapeDtypeStruct((_B,), jnp.float32),
        jax.ShapeDtypeStruct((_B,), jnp.float32),
    ],
    scratch_types=[
        pltpu.VMEM((_BPW,), jnp.int32),
        pltpu.VMEM((_BPW,), jnp.int32),
        pltpu.VMEM((_BPW, _D), jnp.float32),
        pltpu.VMEM((_BPW, _D), jnp.float32),
        pltpu.VMEM((_BPW,), jnp.float32),
        pltpu.VMEM((_BPW,), jnp.float32),
        pltpu.SemaphoreType.DMA,
    ],
    compiler_params=pltpu.CompilerParams(use_tc_tiling_on_sc=False),
)


def _tc_combine_body(feats_ref, u_ref, i_ref, uw_ref, iw_ref,
                     fl_ref, fw_ref, fb_ref, out_ref):
    f = feats_ref[...]
    u = u_ref[...]
    iv = i_ref[...]
    w = fw_ref[...]
    p = lax.dot_general(u, fl_ref[...], (((1,), (1,)), ((), ())),
                        preferred_element_type=jnp.float32)
    lin = jnp.sum(f * w, axis=1, keepdims=True)
    inter1 = jnp.sum(u * iv, axis=1, keepdims=True)
    inter2 = jnp.sum(p * f, axis=1, keepdims=True)
    out_ref[...] = (lin + fb_ref[0, 0] + uw_ref[...] + iw_ref[...]
                    + inter1 + inter2)


def _tc_combine(feats, u_e, i_e, uw, iw, fl, fw, fb):
    nblk = _B // _R
    return pl.pallas_call(
        _tc_combine_body,
        grid=(nblk,),
        in_specs=[
            pl.BlockSpec((_R, _NF), lambda i: (i, 0)),
            pl.BlockSpec((_R, _D), lambda i: (i, 0)),
            pl.BlockSpec((_R, _D), lambda i: (i, 0)),
            pl.BlockSpec((_R, 1), lambda i: (i, 0)),
            pl.BlockSpec((_R, 1), lambda i: (i, 0)),
            pl.BlockSpec((_NF, _D), lambda i: (0, 0)),
            pl.BlockSpec((1, _NF), lambda i: (0, 0)),
            pl.BlockSpec((1, 1), lambda i: (0, 0)),
        ],
        out_specs=pl.BlockSpec((_R, 1), lambda i: (i, 0)),
        out_shape=jax.ShapeDtypeStruct((_B, 1), jnp.float32),
    )(feats, u_e, i_e, uw, iw, fl, fw, fb)


def kernel(x, user_latent, item_latent, feat_latent, fw_W, fw_b,
           user_weight, item_weight):
    users = x[:, 0].astype(jnp.int32)
    items = x[:, 1].astype(jnp.int32) + jnp.int32(1000000)
    feats = x[:, 2:]
    lat = jnp.concatenate([user_latent, item_latent], axis=0)
    wt = jnp.concatenate([jnp.reshape(user_weight, (-1,)),
                          jnp.reshape(item_weight, (-1,))], axis=0)
    u_e, i_e, uw, iw = _sc_gather(lat, wt, users, items)
    return _tc_combine(feats, u_e, i_e,
                       jnp.reshape(uw, (_B, 1)), jnp.reshape(iw, (_B, 1)),
                       feat_latent, fw_W, jnp.reshape(fw_b, (1, 1)))


# speedup vs baseline: 2.1533x; 1.0936x over previous
import jax
import jax.numpy as jnp
from jax import lax
from jax.experimental import pallas as pl
from jax.experimental.pallas import tpu as pltpu
from jax.experimental.pallas import tpu_sc as plsc

_B = 16384
_D = 32
_NF = 26
_NW = 32
_BPW = _B // _NW
_R = 2048


def _sc_gather_body(ul, il, uwt, iwt, uidx, iidx,
                    u_out, i_out, uw_out, iw_out,
                    uidx_v, iidx_v, urows_v, irows_v, uw_v, iw_v, sem):
    wid = lax.axis_index("s") * 2 + lax.axis_index("c")
    base = wid * _BPW
    pltpu.sync_copy(uidx.at[pl.ds(base, _BPW)], uidx_v)
    pltpu.sync_copy(iidx.at[pl.ds(base, _BPW)], iidx_v)
    c0 = pltpu.async_copy(ul.at[uidx_v], urows_v, sem)
    c1 = pltpu.async_copy(il.at[iidx_v], irows_v, sem)
    c2 = pltpu.async_copy(uwt.at[uidx_v], uw_v, sem)
    c3 = pltpu.async_copy(iwt.at[iidx_v], iw_v, sem)
    c0.wait()
    c1.wait()
    c2.wait()
    c3.wait()
    pltpu.sync_copy(urows_v, u_out.at[pl.ds(base, _BPW)])
    pltpu.sync_copy(irows_v, i_out.at[pl.ds(base, _BPW)])
    pltpu.sync_copy(uw_v, uw_out.at[pl.ds(base, _BPW)])
    pltpu.sync_copy(iw_v, iw_out.at[pl.ds(base, _BPW)])


_sc_gather = pl.kernel(
    _sc_gather_body,
    mesh=plsc.VectorSubcoreMesh(core_axis_name="c", subcore_axis_name="s"),
    out_type=[
        jax.ShapeDtypeStruct((_B, _D), jnp.bfloat16),
        jax.ShapeDtypeStruct((_B, _D), jnp.bfloat16),
        jax.ShapeDtypeStruct((_B,), jnp.float32),
        jax.ShapeDtypeStruct((_B,), jnp.float32),
    ],
    scratch_types=[
        pltpu.VMEM((_BPW,), jnp.int32),
        pltpu.VMEM((_BPW,), jnp.int32),
        pltpu.VMEM((_BPW, _D), jnp.bfloat16),
        pltpu.VMEM((_BPW, _D), jnp.bfloat16),
        pltpu.VMEM((_BPW,), jnp.float32),
        pltpu.VMEM((_BPW,), jnp.float32),
        pltpu.SemaphoreType.DMA,
    ],
    compiler_params=pltpu.CompilerParams(use_tc_tiling_on_sc=False),
)


def _tc_combine_body(feats_ref, u_ref, i_ref, uw_ref, iw_ref,
                     fl_ref, fw_ref, fb_ref, out_ref):
    f = feats_ref[...]
    u = u_ref[...].astype(jnp.float32)
    iv = i_ref[...].astype(jnp.float32)
    w = fw_ref[...]
    p = lax.dot_general(u, fl_ref[...], (((1,), (1,)), ((), ())),
                        preferred_element_type=jnp.float32)
    lin = jnp.sum(f * w, axis=1, keepdims=True)
    inter1 = jnp.sum(u * iv, axis=1, keepdims=True)
    inter2 = jnp.sum(p * f, axis=1, keepdims=True)
    out_ref[...] = (lin + fb_ref[0, 0] + uw_ref[...] + iw_ref[...]
                    + inter1 + inter2)


def _tc_combine(feats, u_e, i_e, uw, iw, fl, fw, fb):
    nblk = _B // _R
    return pl.pallas_call(
        _tc_combine_body,
        grid=(nblk,),
        in_specs=[
            pl.BlockSpec((_R, _NF), lambda i: (i, 0)),
            pl.BlockSpec((_R, _D), lambda i: (i, 0)),
            pl.BlockSpec((_R, _D), lambda i: (i, 0)),
            pl.BlockSpec((_R, 1), lambda i: (i, 0)),
            pl.BlockSpec((_R, 1), lambda i: (i, 0)),
            pl.BlockSpec((_NF, _D), lambda i: (0, 0)),
            pl.BlockSpec((1, _NF), lambda i: (0, 0)),
            pl.BlockSpec((1, 1), lambda i: (0, 0)),
        ],
        out_specs=pl.BlockSpec((_R, 1), lambda i: (i, 0)),
        out_shape=jax.ShapeDtypeStruct((_B, 1), jnp.float32),
    )(feats, u_e, i_e, uw, iw, fl, fw, fb)


def kernel(x, user_latent, item_latent, feat_latent, fw_W, fw_b,
           user_weight, item_weight):
    users = x[:, 0].astype(jnp.int32)
    items = x[:, 1].astype(jnp.int32)
    feats = x[:, 2:]
    ul16 = user_latent.astype(jnp.bfloat16)
    il16 = item_latent.astype(jnp.bfloat16)
    uwt = jnp.reshape(user_weight, (-1,))
    iwt = jnp.reshape(item_weight, (-1,))
    u_e, i_e, uw, iw = _sc_gather(ul16, il16, uwt, iwt, users, items)
    return _tc_combine(feats, u_e, i_e,
                       jnp.reshape(uw, (_B, 1)), jnp.reshape(iw, (_B, 1)),
                       feat_latent, fw_W, jnp.reshape(fw_b, (1, 1)))


# speedup vs baseline: 2.5207x; 1.1706x over previous
import jax
import jax.numpy as jnp
from jax import lax
from jax.experimental import pallas as pl
from jax.experimental.pallas import tpu as pltpu
from jax.experimental.pallas import tpu_sc as plsc

_B = 16384
_D = 32
_NF = 26
_NW = 32
_BPW = _B // _NW
_R = 2048


def _sc_gather_body(ul, il, uwt, iwt, uidx, iidx,
                    u_out, i_out, uw_out, iw_out,
                    uidx_v, iidx_v, urows_v, irows_v, uw_v, iw_v, sem):
    wid = lax.axis_index("s") * 2 + lax.axis_index("c")
    base = wid * _BPW
    pltpu.sync_copy(uidx.at[pl.ds(base, _BPW)], uidx_v)
    pltpu.sync_copy(iidx.at[pl.ds(base, _BPW)], iidx_v)
    c0 = pltpu.async_copy(ul.at[uidx_v], urows_v, sem)
    c1 = pltpu.async_copy(il.at[iidx_v], irows_v, sem)
    c2 = pltpu.async_copy(uwt.at[uidx_v], uw_v, sem)
    c3 = pltpu.async_copy(iwt.at[iidx_v], iw_v, sem)
    c0.wait()
    c1.wait()
    c2.wait()
    c3.wait()
    pltpu.sync_copy(urows_v, u_out.at[pl.ds(base, _BPW)])
    pltpu.sync_copy(irows_v, i_out.at[pl.ds(base, _BPW)])
    pltpu.sync_copy(uw_v, uw_out.at[pl.ds(base, _BPW)])
    pltpu.sync_copy(iw_v, iw_out.at[pl.ds(base, _BPW)])


_sc_gather = pl.kernel(
    _sc_gather_body,
    mesh=plsc.VectorSubcoreMesh(core_axis_name="c", subcore_axis_name="s"),
    out_type=[
        jax.ShapeDtypeStruct((_B, _D), jnp.float32),
        jax.ShapeDtypeStruct((_B, _D), jnp.float32),
        jax.ShapeDtypeStruct((_B,), jnp.float32),
        jax.ShapeDtypeStruct((_B,), jnp.float32),
    ],
    scratch_types=[
        pltpu.VMEM((_BPW,), jnp.int32),
        pltpu.VMEM((_BPW,), jnp.int32),
        pltpu.VMEM((_BPW, _D), jnp.float32),
        pltpu.VMEM((_BPW, _D), jnp.float32),
        pltpu.VMEM((_BPW,), jnp.float32),
        pltpu.VMEM((_BPW,), jnp.float32),
        pltpu.SemaphoreType.DMA,
    ],
    compiler_params=pltpu.CompilerParams(use_tc_tiling_on_sc=False),
)


def _tc_combine_body(feats_ref, u_ref, i_ref, uw_ref, iw_ref,
                     fl_ref, fw_ref, fb_ref, out_ref):
    f = feats_ref[...]
    u = u_ref[...]
    iv = i_ref[...]
    w = fw_ref[...]
    p = lax.dot_general(u, fl_ref[...], (((1,), (1,)), ((), ())),
                        preferred_element_type=jnp.float32)
    lin = jnp.sum(f * w, axis=1, keepdims=True)
    inter1 = jnp.sum(u * iv, axis=1, keepdims=True)
    inter2 = jnp.sum(p * f, axis=1, keepdims=True)
    out_ref[...] = (lin + fb_ref[0, 0] + uw_ref[...] + iw_ref[...]
                    + inter1 + inter2)


def _tc_combine(feats, u_e, i_e, uw, iw, fl, fw, fb):
    nblk = _B // _R
    return pl.pallas_call(
        _tc_combine_body,
        grid=(nblk,),
        in_specs=[
            pl.BlockSpec((_R, _NF), lambda i: (i, 0)),
            pl.BlockSpec((_R, _D), lambda i: (i, 0)),
            pl.BlockSpec((_R, _D), lambda i: (i, 0)),
            pl.BlockSpec((_R, 1), lambda i: (i, 0)),
            pl.BlockSpec((_R, 1), lambda i: (i, 0)),
            pl.BlockSpec((_NF, _D), lambda i: (0, 0)),
            pl.BlockSpec((1, _NF), lambda i: (0, 0)),
            pl.BlockSpec((1, 1), lambda i: (0, 0)),
        ],
        out_specs=pl.BlockSpec((_R, 1), lambda i: (i, 0)),
        out_shape=jax.ShapeDtypeStruct((_B, 1), jnp.float32),
    )(feats, u_e, i_e, uw, iw, fl, fw, fb)


def kernel(x, user_latent, item_latent, feat_latent, fw_W, fw_b,
           user_weight, item_weight):
    users = x[:, 0].astype(jnp.int32)
    items = x[:, 1].astype(jnp.int32)
    feats = x[:, 2:]
    uwt = jnp.reshape(user_weight, (-1,))
    iwt = jnp.reshape(item_weight, (-1,))
    u_e, i_e, uw, iw = _sc_gather(user_latent, item_latent, uwt, iwt,
                                  users, items)
    return _tc_combine(feats, u_e, i_e,
                       jnp.reshape(uw, (_B, 1)), jnp.reshape(iw, (_B, 1)),
                       feat_latent, fw_W, jnp.reshape(fw_b, (1, 1)))
